# Initial kernel scaffold; baseline (speedup 1.0000x reference)
import jax
import jax.numpy as jnp
from jax.experimental import pallas as pl


def kernel(x, edge_index, batch, W1_rel, b1_rel, W1_root, W2_rel, b2_rel, W2_root):
    raise NotImplementedError("write your pallas kernel here")


# speedup vs baseline: 12.5382x; 12.5382x over previous
import functools

import jax
import jax.numpy as jnp
from jax import lax
from jax.experimental import pallas as pl
from jax.experimental.pallas import tpu as pltpu
from jax.experimental.pallas import tpu_sc as plsc

N = 10000
FEAT = 128
G = 64
NC = 2
NS = 16
NW = NC * NS
CHUNK = 128
N_PAD = 10112
RPT = N_PAD // NS


def _sc_agg(K):
  mesh = plsc.VectorSubcoreMesh(core_axis_name="c", subcore_axis_name="s")

  NPASS = 2
  HK = K // NPASS

  def body(src_hbm, dst_hbm, xr_hbm, zeros_hbm, out_hbm,
           shared, src_v, dst_v, rows_a, rows_b, sem_a, sem_b):
    c = lax.axis_index("c")
    s = lax.axis_index("s")
    wid = c * NS + s
    pltpu.sync_copy(zeros_hbm.at[pl.ds(s * RPT, RPT)],
                    shared.at[pl.ds(s * RPT, RPT)])
    plsc.subcore_barrier()

    for p in range(NPASS):
      base = wid * K + p * HK
      pltpu.sync_copy(src_hbm.at[pl.ds(base, HK)], src_v)
      pltpu.sync_copy(dst_hbm.at[pl.ds(base, HK)], dst_v)
      pltpu.async_copy(xr_hbm.at[src_v.at[0]], rows_a, sem_a)

      def step(i, carry):
        j0 = 2 * i
        pltpu.async_copy(xr_hbm.at[src_v.at[j0 + 1]], rows_b, sem_b)
        pltpu.make_async_copy(xr_hbm.at[src_v.at[j0]], rows_a, sem_a).wait()
        pltpu.sync_copy(rows_a, shared.at[dst_v.at[j0]], add=True)
        jn = lax.min(j0 + 2, HK - 1)
        pltpu.async_copy(xr_hbm.at[src_v.at[jn]], rows_a, sem_a)
        pltpu.make_async_copy(xr_hbm.at[src_v.at[j0 + 1]], rows_b, sem_b
                              ).wait()
        pltpu.sync_copy(rows_b, shared.at[dst_v.at[j0 + 1]], add=True)
        return carry

      lax.fori_loop(0, HK // 2, step, 0)
      pltpu.make_async_copy(xr_hbm.at[src_v.at[HK - 1]], rows_a, sem_a).wait()

    plsc.subcore_barrier()
    pltpu.sync_copy(shared.at[pl.ds(s * RPT, RPT)],
                    out_hbm.at[pl.ds(c * N_PAD + s * RPT, RPT)])

  return pl.kernel(
      body,
      mesh=mesh,
      out_type=jax.ShapeDtypeStruct((NC * N_PAD, FEAT), jnp.float32),
      scratch_types=[
          pltpu.VMEM_SHARED((N_PAD, FEAT), jnp.float32),
          pltpu.VMEM((HK, CHUNK), jnp.int32),
          pltpu.VMEM((HK, CHUNK), jnp.int32),
          pltpu.VMEM((CHUNK, FEAT), jnp.float32),
          pltpu.VMEM((CHUNK, FEAT), jnp.float32),
          pltpu.SemaphoreType.DMA,
          pltpu.SemaphoreType.DMA,
      ],
  )


def _mm_t(a, w_ref):
  return lax.dot_general(a, w_ref[...], (((1,), (1,)), ((), ())),
                         preferred_element_type=jnp.float32)


def _tc_pre(x_ref, w_ref, o_ref):
  o_ref[...] = _mm_t(x_ref[...], w_ref)


def _tc_mid(p_ref, x_ref, wroot_ref, wrel2_ref, b1_ref, x1_ref, xr2_ref):
  p = p_ref[pl.ds(0, N), :] + p_ref[pl.ds(N_PAD, N), :]
  x1 = jnp.tanh(p + b1_ref[...] + _mm_t(x_ref[...], wroot_ref))
  x1_ref[...] = x1
  xr2_ref[...] = _mm_t(x1, wrel2_ref)


def _tc_post(p_ref, x1_ref, wroot2_ref, b2_ref, batch_ref, o_ref):
  p = p_ref[pl.ds(0, N), :] + p_ref[pl.ds(N_PAD, N), :]
  x2 = jnp.tanh(p + b2_ref[...] + _mm_t(x1_ref[...], wroot2_ref))
  onehot = (batch_ref[...] == lax.broadcasted_iota(jnp.int32, (N, G), 1)
            ).astype(jnp.float32)
  sums = lax.dot_general(onehot, x2, (((0,), (0,)), ((), ())),
                         preferred_element_type=jnp.float32)
  counts = jnp.sum(onehot, axis=0)
  o_ref[...] = sums / jnp.maximum(counts, 1.0)[:, None]


def kernel(x, edge_index, batch, W1_rel, b1_rel, W1_root, W2_rel, b2_rel,
           W2_root):
  src = edge_index[0]
  dst = edge_index[1]
  E = src.shape[0]
  K = -(-E // (NW * CHUNK))
  K = -(-K // 8) * 8
  pad = NW * CHUNK * K - E
  pad_ar = jnp.arange(pad, dtype=jnp.int32)
  src_p = jnp.concatenate(
      [src, pad_ar % N]).reshape(NW * K, CHUNK)
  dst_p = jnp.concatenate(
      [dst, N + pad_ar % (N_PAD - N)]).reshape(NW * K, CHUNK)
  zeros = jnp.zeros((N_PAD, FEAT), jnp.float32)

  xr1 = pl.pallas_call(
      _tc_pre,
      out_shape=jax.ShapeDtypeStruct((N, FEAT), jnp.float32),
  )(x, W1_rel)

  agg1 = _sc_agg(K)(src_p, dst_p, xr1, zeros)

  x1, xr2 = pl.pallas_call(
      _tc_mid,
      out_shape=[jax.ShapeDtypeStruct((N, FEAT), jnp.float32)] * 2,
  )(agg1, x, W1_root, W2_rel, b1_rel.reshape(1, FEAT))

  agg2 = _sc_agg(K)(src_p, dst_p, xr2, zeros)

  xout = pl.pallas_call(
      _tc_post,
      out_shape=jax.ShapeDtypeStruct((G, FEAT), jnp.float32),
  )(agg2, x1, W2_root, b2_rel.reshape(1, FEAT), batch.reshape(N, 1))
  return xout


# speedup vs baseline: 12.5617x; 1.0019x over previous
import functools

import jax
import jax.numpy as jnp
from jax import lax
from jax.experimental import pallas as pl
from jax.experimental.pallas import tpu as pltpu
from jax.experimental.pallas import tpu_sc as plsc

N = 10000
FEAT = 128
G = 64
NC = 2
NS = 16
NW = NC * NS
CHUNK = 128
N_PAD = 10112
RPT = N_PAD // NS


def _sc_agg(K):
  mesh = plsc.VectorSubcoreMesh(core_axis_name="c", subcore_axis_name="s")

  NPASS = 2
  HK = K // NPASS

  def body(src_hbm, dst_hbm, xr_hbm, zeros_hbm, out_hbm,
           shared, src_v, dst_v, rows_a, rows_b, sem_a, sem_b, sem_c):
    c = lax.axis_index("c")
    s = lax.axis_index("s")
    wid = c * NS + s
    pltpu.async_copy(zeros_hbm.at[pl.ds(s * RPT, RPT)],
                     shared.at[pl.ds(s * RPT, RPT)], sem_c)
    pltpu.async_copy(src_hbm.at[pl.ds(wid * K * CHUNK, HK * CHUNK)], src_v,
                     sem_a)
    pltpu.async_copy(dst_hbm.at[pl.ds(wid * K, HK)], dst_v, sem_b)
    pltpu.make_async_copy(src_hbm.at[pl.ds(0, HK * CHUNK)], src_v,
                          sem_a).wait()
    pltpu.make_async_copy(dst_hbm.at[pl.ds(0, HK)], dst_v, sem_b).wait()
    pltpu.make_async_copy(zeros_hbm.at[pl.ds(0, RPT)],
                          shared.at[pl.ds(0, RPT)], sem_c).wait()
    plsc.subcore_barrier()

    for p in range(NPASS):
      if p > 0:
        base = wid * K + p * HK
        pltpu.sync_copy(src_hbm.at[pl.ds(base * CHUNK, HK * CHUNK)], src_v)
        pltpu.sync_copy(dst_hbm.at[pl.ds(base, HK)], dst_v)

      def sidx(j):
        return src_v.at[pl.ds(j * CHUNK, CHUNK)]

      pltpu.async_copy(xr_hbm.at[sidx(0)], rows_a, sem_a)

      def step(i, carry):
        j0 = 2 * i
        pltpu.async_copy(xr_hbm.at[sidx(j0 + 1)], rows_b, sem_b)
        pltpu.make_async_copy(xr_hbm.at[sidx(j0)], rows_a, sem_a).wait()
        pltpu.sync_copy(rows_a, shared.at[dst_v.at[j0]], add=True)
        jn = lax.min(j0 + 2, HK - 1)
        pltpu.async_copy(xr_hbm.at[sidx(jn)], rows_a, sem_a)
        pltpu.make_async_copy(xr_hbm.at[sidx(j0 + 1)], rows_b, sem_b
                              ).wait()
        pltpu.sync_copy(rows_b, shared.at[dst_v.at[j0 + 1]], add=True)
        return carry

      lax.fori_loop(0, HK // 2, step, 0)
      pltpu.make_async_copy(xr_hbm.at[sidx(HK - 1)], rows_a, sem_a).wait()

    plsc.subcore_barrier()
    pltpu.sync_copy(shared.at[pl.ds(s * RPT, RPT)],
                    out_hbm.at[pl.ds(c * N_PAD + s * RPT, RPT)])

  return pl.kernel(
      body,
      mesh=mesh,
      out_type=jax.ShapeDtypeStruct((NC * N_PAD, FEAT), jnp.float32),
      scratch_types=[
          pltpu.VMEM_SHARED((N_PAD, FEAT), jnp.float32),
          pltpu.VMEM((HK * CHUNK,), jnp.int32),
          pltpu.VMEM((HK, CHUNK), jnp.int32),
          pltpu.VMEM((CHUNK, FEAT), jnp.float32),
          pltpu.VMEM((CHUNK, FEAT), jnp.float32),
          pltpu.SemaphoreType.DMA,
          pltpu.SemaphoreType.DMA,
          pltpu.SemaphoreType.DMA,
      ],
  )


def _mm_t(a, w_ref):
  return lax.dot_general(a, w_ref[...], (((1,), (1,)), ((), ())),
                         preferred_element_type=jnp.float32)


def _tc_pre(x_ref, w_ref, o_ref):
  o_ref[...] = _mm_t(x_ref[...], w_ref)


def _tc_mid(p_ref, x_ref, wroot_ref, wrel2_ref, b1_ref, x1_ref, xr2_ref):
  p = p_ref[pl.ds(0, N), :] + p_ref[pl.ds(N_PAD, N), :]
  x1 = jnp.tanh(p + b1_ref[...] + _mm_t(x_ref[...], wroot_ref))
  x1_ref[...] = x1
  xr2_ref[...] = _mm_t(x1, wrel2_ref)


def _tc_post(p_ref, x1_ref, wroot2_ref, b2_ref, batch_ref, o_ref):
  p = p_ref[pl.ds(0, N), :] + p_ref[pl.ds(N_PAD, N), :]
  x2 = jnp.tanh(p + b2_ref[...] + _mm_t(x1_ref[...], wroot2_ref))
  onehot = (batch_ref[...] == lax.broadcasted_iota(jnp.int32, (N, G), 1)
            ).astype(jnp.float32)
  sums = lax.dot_general(onehot, x2, (((0,), (0,)), ((), ())),
                         preferred_element_type=jnp.float32)
  counts = jnp.sum(onehot, axis=0)
  o_ref[...] = sums / jnp.maximum(counts, 1.0)[:, None]


def kernel(x, edge_index, batch, W1_rel, b1_rel, W1_root, W2_rel, b2_rel,
           W2_root):
  src = edge_index[0]
  dst = edge_index[1]
  E = src.shape[0]
  K = -(-E // (NW * CHUNK))
  K = -(-K // 8) * 8
  pad = NW * CHUNK * K - E
  pad_ar = jnp.arange(pad, dtype=jnp.int32)
  src_p = jnp.concatenate([src, pad_ar % N])
  dst_p = jnp.concatenate(
      [dst, N + pad_ar % (N_PAD - N)]).reshape(NW * K, CHUNK)
  zeros = jnp.zeros((N_PAD, FEAT), jnp.float32)

  xr1 = pl.pallas_call(
      _tc_pre,
      out_shape=jax.ShapeDtypeStruct((N, FEAT), jnp.float32),
  )(x, W1_rel)

  agg1 = _sc_agg(K)(src_p, dst_p, xr1, zeros)

  x1, xr2 = pl.pallas_call(
      _tc_mid,
      out_shape=[jax.ShapeDtypeStruct((N, FEAT), jnp.float32)] * 2,
  )(agg1, x, W1_root, W2_rel, b1_rel.reshape(1, FEAT))

  agg2 = _sc_agg(K)(src_p, dst_p, xr2, zeros)

  xout = pl.pallas_call(
      _tc_post,
      out_shape=jax.ShapeDtypeStruct((G, FEAT), jnp.float32),
  )(agg2, x1, W2_root, b2_rel.reshape(1, FEAT), batch.reshape(N, 1))
  return xout
